# baseline (speedup 1.0000x reference)
import functools

import jax
import jax.numpy as jnp
from jax import lax
from jax.experimental import pallas as pl
from jax.experimental.pallas import tpu as pltpu
from jax.experimental.pallas import tpu_sc as plsc

F32 = jnp.float32

NC = 2
NS = 16
LANE = 128


def _mesh():
    return plsc.VectorSubcoreMesh(
        core_axis_name="c", subcore_axis_name="s", num_cores=NC, num_subcores=NS
    )


def _make_deg(n_pad, e_rows):
    rows_per_tile = e_rows // (NC * NS)
    n_per_tile = n_pad // NS

    @functools.partial(
        pl.kernel,
        out_type=(
            jax.ShapeDtypeStruct((n_pad,), F32),
            jax.ShapeDtypeStruct((n_pad,), F32),
        ),
        mesh=_mesh(),
        scratch_types=[
            pltpu.VMEM_SHARED((n_pad,), F32),
            pltpu.VMEM((rows_per_tile, LANE), jnp.int32),
            pltpu.VMEM((LANE,), F32),
            pltpu.VMEM((n_per_tile,), F32),
        ],
    )
    def deg_kernel(dst2d, zeros1, out0, out1, acc, idx_v, ones_v, wb_v):
        c = lax.axis_index("c")
        s = lax.axis_index("s")
        pltpu.sync_copy(
            zeros1.at[pl.ds(s * n_per_tile, n_per_tile)],
            acc.at[pl.ds(s * n_per_tile, n_per_tile)],
        )
        for i in range(LANE // 16):
            ones_v[pl.ds(i * 16, 16)] = jnp.full((16,), 1.0, F32)
        row0 = c * (e_rows // NC) + s * rows_per_tile
        pltpu.sync_copy(dst2d.at[pl.ds(row0, rows_per_tile)], idx_v)
        plsc.subcore_barrier()

        def body(j, _):
            pltpu.sync_copy(ones_v, acc.at[idx_v.at[j]], add=True)
            return 0

        lax.fori_loop(0, rows_per_tile, body, 0)
        plsc.subcore_barrier()
        sl = pl.ds(s * n_per_tile, n_per_tile)
        pltpu.sync_copy(acc.at[sl], wb_v)

        @pl.when(c == 0)
        def _():
            pltpu.sync_copy(wb_v, out0.at[sl])

        @pl.when(c == 1)
        def _():
            pltpu.sync_copy(wb_v, out1.at[sl])

    return deg_kernel


CH = 64
G = 4


def _make_agg(n_pad, e_rows, half):
    rows_per_tile = e_rows // NS
    ngroups = rows_per_tile // G
    assert rows_per_tile % G == 0 and ngroups % 2 == 0
    n_per_tile = n_pad // NS
    wb_chunks = n_per_tile // CH

    @functools.partial(
        pl.kernel,
        out_type=(
            jax.ShapeDtypeStruct((n_pad, half), F32),
            jax.ShapeDtypeStruct((n_pad, half), F32),
        ),
        mesh=_mesh(),
        scratch_types=[
            pltpu.VMEM_SHARED((n_pad, half), F32),
            pltpu.VMEM((2 * G, 2, CH), jnp.int32),
            [pltpu.VMEM((CH, half), F32) for _ in range(G)],
            [pltpu.SemaphoreType.DMA for _ in range(2)],
            [pltpu.SemaphoreType.DMA for _ in range(G)],
            [pltpu.SemaphoreType.DMA for _ in range(G)],
        ],
    )
    def agg_kernel(edg3d, tab_lo, tab_hi, zeros2,
                   out_lo, out_hi, acc, idxb, rows_v, isems, gsems, ssems):
        c = lax.axis_index("c")
        s = lax.axis_index("s")
        nsl = pl.ds(s * n_per_tile, n_per_tile)
        pltpu.sync_copy(zeros2.at[nsl], acc.at[nsl])
        plsc.subcore_barrier()
        row0 = s * rows_per_tile

        def run(tab, out):
            def prefetch(g, bank):
                pltpu.async_copy(edg3d.at[pl.ds(row0 + g * G, G)],
                                 idxb.at[pl.ds(bank * G, G)], isems[bank])

            def wait_idx(bank):
                pltpu.make_async_copy(edg3d.at[pl.ds(row0, G)],
                                      idxb.at[pl.ds(bank * G, G)],
                                      isems[bank]).wait()

            def gather(bank, b):
                pltpu.async_copy(tab.at[idxb.at[bank * G + b, 0]],
                                 rows_v[b], gsems[b])

            def wait_gather(b):
                pltpu.make_async_copy(tab.at[idxb.at[0, 0]], rows_v[b],
                                      gsems[b]).wait()

            def scatter(bank, b):
                pltpu.async_copy(rows_v[b], acc.at[idxb.at[bank * G + b, 1]],
                                 ssems[b], add=True)

            def wait_scatter(b):
                pltpu.make_async_copy(rows_v[b], acc.at[idxb.at[0, 1]],
                                      ssems[b]).wait()

            prefetch(0, 0)
            prefetch(1, 1)
            wait_idx(0)
            for b in range(G):
                gather(0, b)

            def super_body(si, _):
                for bank in range(2):
                    g = 2 * si + bank
                    for b in range(G):
                        wait_gather(b)
                    for b in range(G):
                        scatter(bank, b)
                    for b in range(G):
                        wait_scatter(b)
                    pl.when(g + 2 < ngroups)(
                        lambda g=g, bank=bank: prefetch(g + 2, bank))

                    def next_gathers(bank=bank):
                        wait_idx(1 - bank)
                        for b in range(G):
                            gather(1 - bank, b)

                    pl.when(g + 1 < ngroups)(next_gathers)
                return 0

            lax.fori_loop(0, ngroups // 2, super_body, 0)
            plsc.subcore_barrier()
            for q in range(wb_chunks):
                sl = pl.ds(s * n_per_tile + q * CH, CH)
                pltpu.sync_copy(acc.at[sl], rows_v[0])
                pltpu.sync_copy(rows_v[0], out.at[sl])

        @pl.when(c == 0)
        def _():
            run(tab_lo, out_lo)

        @pl.when(c == 1)
        def _():
            run(tab_hi, out_hi)

    return agg_kernel


def _tc_scale(deg0, deg1, x_pad, half):
    n_pad, fin = x_pad.shape
    blk = 1024
    grid = (n_pad // blk,)

    def body(d0, d1, x, lo, hi):
        dinv = lax.rsqrt(d0[...] + d1[...] + 1.0)
        xs = x[...] * dinv[:, None]
        lo[...] = xs[:, :half]
        hi[...] = xs[:, half:]

    return pl.pallas_call(
        body,
        grid=grid,
        in_specs=[
            pl.BlockSpec((blk,), lambda i: (i,)),
            pl.BlockSpec((blk,), lambda i: (i,)),
            pl.BlockSpec((blk, fin), lambda i: (i, 0)),
        ],
        out_specs=[
            pl.BlockSpec((blk, half), lambda i: (i, 0)),
            pl.BlockSpec((blk, half), lambda i: (i, 0)),
        ],
        out_shape=[
            jax.ShapeDtypeStruct((n_pad, half), F32),
            jax.ShapeDtypeStruct((n_pad, half), F32),
        ],
    )(deg0, deg1, x_pad)


def _tc_layer1(deg0, deg1, x_pad, u_lo, u_hi, W1, b1, W2, half):
    n_pad, fin = x_pad.shape
    fmid = W1.shape[1]
    blk = 1024
    grid = (n_pad // blk,)

    def body(d0, d1, x, ulo, uhi, w1, bb1, w2, t_out, tslo, tshi):
        dinv = lax.rsqrt(d0[...] + d1[...] + 1.0)
        u = jnp.concatenate([ulo[...], uhi[...]], axis=1)
        z = u * dinv[:, None] + x[...] * (dinv * dinv)[:, None]
        h = jnp.maximum(
            jnp.dot(z, w1[...], preferred_element_type=F32) + bb1[...][None, :],
            0.0,
        )
        t = jnp.dot(h, w2[...], preferred_element_type=F32)
        t_out[...] = t
        ts = t * dinv[:, None]
        tslo[...] = ts[:, :half]
        tshi[...] = ts[:, half:]

    return pl.pallas_call(
        body,
        grid=grid,
        in_specs=[
            pl.BlockSpec((blk,), lambda i: (i,)),
            pl.BlockSpec((blk,), lambda i: (i,)),
            pl.BlockSpec((blk, fin), lambda i: (i, 0)),
            pl.BlockSpec((blk, half), lambda i: (i, 0)),
            pl.BlockSpec((blk, half), lambda i: (i, 0)),
            pl.BlockSpec((fin, fmid), lambda i: (0, 0)),
            pl.BlockSpec((fmid,), lambda i: (0,)),
            pl.BlockSpec((fmid, fin), lambda i: (0, 0)),
        ],
        out_specs=[
            pl.BlockSpec((blk, fin), lambda i: (i, 0)),
            pl.BlockSpec((blk, half), lambda i: (i, 0)),
            pl.BlockSpec((blk, half), lambda i: (i, 0)),
        ],
        out_shape=[
            jax.ShapeDtypeStruct((n_pad, fin), F32),
            jax.ShapeDtypeStruct((n_pad, half), F32),
            jax.ShapeDtypeStruct((n_pad, half), F32),
        ],
    )(deg0, deg1, x_pad, u_lo, u_hi, W1, b1, W2)


def _tc_finish(deg0, deg1, t, u_lo, u_hi, b2):
    n_pad, fout = t.shape
    half = fout // 2
    blk = 1024
    grid = (n_pad // blk,)

    def body(d0, d1, tt, ulo, uhi, bb2, out):
        dinv = lax.rsqrt(d0[...] + d1[...] + 1.0)
        u = jnp.concatenate([ulo[...], uhi[...]], axis=1)
        z = u * dinv[:, None] + tt[...] * (dinv * dinv)[:, None] + bb2[...][None, :]
        r = jnp.maximum(z, 0.0)
        m = jnp.max(r, axis=1, keepdims=True)
        lse = m + jnp.log(jnp.sum(jnp.exp(r - m), axis=1, keepdims=True))
        out[...] = r - lse

    return pl.pallas_call(
        body,
        grid=grid,
        in_specs=[
            pl.BlockSpec((blk,), lambda i: (i,)),
            pl.BlockSpec((blk,), lambda i: (i,)),
            pl.BlockSpec((blk, fout), lambda i: (i, 0)),
            pl.BlockSpec((blk, half), lambda i: (i, 0)),
            pl.BlockSpec((blk, half), lambda i: (i, 0)),
            pl.BlockSpec((fout,), lambda i: (0,)),
        ],
        out_specs=pl.BlockSpec((blk, fout), lambda i: (i, 0)),
        out_shape=jax.ShapeDtypeStruct((n_pad, fout), F32),
    )(deg0, deg1, t, u_lo, u_hi, b2)


def kernel(x, edge_index, W1, b1, W2, b2):
    n, fin = x.shape
    half = fin // 2
    e = edge_index.shape[1]

    n_pad = ((n + 1 + 1023) // 1024) * 1024
    e_pad = ((e + NC * NS * LANE - 1) // (NC * NS * LANE)) * (NC * NS * LANE)

    ei = edge_index.astype(jnp.int32)
    pad = jnp.full((e_pad - e,), n, jnp.int32)
    src2d = jnp.concatenate([ei[0], pad]).reshape(e_pad // LANE, LANE)
    dst2d = jnp.concatenate([ei[1], pad]).reshape(e_pad // LANE, LANE)
    edg3d = jnp.stack(
        [src2d.reshape(e_pad // CH, CH), dst2d.reshape(e_pad // CH, CH)],
        axis=1,
    )
    x_pad = jnp.pad(x, ((0, n_pad - n), (0, 0)))
    zeros1 = jnp.zeros((n_pad,), F32)
    zeros2 = jnp.zeros((n_pad, half), F32)

    deg0, deg1 = _make_deg(n_pad, e_pad // LANE)(dst2d, zeros1)
    xs_lo, xs_hi = _tc_scale(deg0, deg1, x_pad, half)
    agg = _make_agg(n_pad, e_pad // CH, half)
    u1_lo, u1_hi = agg(edg3d, xs_lo, xs_hi, zeros2)
    t, ts_lo, ts_hi = _tc_layer1(deg0, deg1, x_pad, u1_lo, u1_hi, W1, b1, W2, half)
    u2_lo, u2_hi = agg(edg3d, ts_lo, ts_hi, zeros2)
    o = _tc_finish(deg0, deg1, t, u2_lo, u2_hi, b2)
    return o[:n]


# speedup vs baseline: 1.1921x; 1.1921x over previous
import functools

import jax
import jax.numpy as jnp
from jax import lax
from jax.experimental import pallas as pl
from jax.experimental.pallas import tpu as pltpu
from jax.experimental.pallas import tpu_sc as plsc

F32 = jnp.float32

NC = 2
NS = 16
LANE = 128


def _mesh():
    return plsc.VectorSubcoreMesh(
        core_axis_name="c", subcore_axis_name="s", num_cores=NC, num_subcores=NS
    )


def _make_deg(n_pad, e_rows):
    rows_per_tile = e_rows // (NC * NS)
    n_per_tile = n_pad // NS

    @functools.partial(
        pl.kernel,
        out_type=(
            jax.ShapeDtypeStruct((n_pad,), F32),
            jax.ShapeDtypeStruct((n_pad,), F32),
        ),
        mesh=_mesh(),
        scratch_types=[
            pltpu.VMEM_SHARED((n_pad,), F32),
            pltpu.VMEM((rows_per_tile, LANE), jnp.int32),
            pltpu.VMEM((LANE,), F32),
            pltpu.VMEM((n_per_tile,), F32),
        ],
    )
    def deg_kernel(dst2d, zeros1, out0, out1, acc, idx_v, ones_v, wb_v):
        c = lax.axis_index("c")
        s = lax.axis_index("s")
        pltpu.sync_copy(
            zeros1.at[pl.ds(s * n_per_tile, n_per_tile)],
            acc.at[pl.ds(s * n_per_tile, n_per_tile)],
        )
        for i in range(LANE // 16):
            ones_v[pl.ds(i * 16, 16)] = jnp.full((16,), 1.0, F32)
        row0 = c * (e_rows // NC) + s * rows_per_tile
        pltpu.sync_copy(dst2d.at[pl.ds(row0, rows_per_tile)], idx_v)
        plsc.subcore_barrier()

        def body(j, _):
            pltpu.sync_copy(ones_v, acc.at[idx_v.at[j]], add=True)
            return 0

        lax.fori_loop(0, rows_per_tile, body, 0)
        plsc.subcore_barrier()
        sl = pl.ds(s * n_per_tile, n_per_tile)
        pltpu.sync_copy(acc.at[sl], wb_v)

        @pl.when(c == 0)
        def _():
            pltpu.sync_copy(wb_v, out0.at[sl])

        @pl.when(c == 1)
        def _():
            pltpu.sync_copy(wb_v, out1.at[sl])

    return deg_kernel


AGG_DT = jnp.float32


def _make_agg(n_pad, e_rows, half):
    rows_per_tile = e_rows // NS
    n_per_tile = n_pad // NS
    wb_chunks = n_per_tile // LANE

    nib = 4
    ngb = 2
    assert rows_per_tile % nib == 0

    @functools.partial(
        pl.kernel,
        out_type=(
            jax.ShapeDtypeStruct((n_pad, half), AGG_DT),
            jax.ShapeDtypeStruct((n_pad, half), AGG_DT),
        ),
        mesh=_mesh(),
        scratch_types=[
            pltpu.VMEM_SHARED((n_pad, half), AGG_DT),
            [pltpu.VMEM((2, LANE), jnp.int32) for _ in range(nib)],
            [pltpu.VMEM((LANE, half), AGG_DT) for _ in range(ngb)],
            [pltpu.SemaphoreType.DMA for _ in range(nib)],
            [pltpu.SemaphoreType.DMA for _ in range(ngb)],
        ],
    )
    def agg_kernel(edg3d, tab_lo, tab_hi, zeros2,
                   out_lo, out_hi, acc, idx_v, rows_v, isems, gsems):
        c = lax.axis_index("c")
        s = lax.axis_index("s")
        nsl = pl.ds(s * n_per_tile, n_per_tile)
        pltpu.sync_copy(zeros2.at[nsl], acc.at[nsl])
        plsc.subcore_barrier()
        row0 = s * rows_per_tile

        def run(tab, out):
            def prefetch(j, ib):
                pltpu.async_copy(edg3d.at[row0 + j], idx_v[ib], isems[ib])

            def wait_idx(ib):
                pltpu.make_async_copy(edg3d.at[row0], idx_v[ib],
                                      isems[ib]).wait()

            def gather(ib, gb):
                pltpu.async_copy(tab.at[idx_v[ib].at[0]], rows_v[gb],
                                 gsems[gb])

            def wait_gather(gb):
                pltpu.make_async_copy(tab.at[idx_v[0].at[0]], rows_v[gb],
                                      gsems[gb]).wait()

            for j in range(nib):
                prefetch(j, j)
            for j in range(ngb):
                wait_idx(j)
                gather(j, j)

            def outer(i, _):
                for b in range(nib):
                    j = i * nib + b
                    gb = b % ngb
                    ib2 = (b + ngb) % nib
                    wait_gather(gb)
                    pltpu.sync_copy(rows_v[gb], acc.at[idx_v[b].at[1]],
                                    add=True)
                    pl.when(j + nib < rows_per_tile)(
                        lambda j=j, b=b: prefetch(j + nib, b))

                    def nxt_gather(ib2=ib2, gb=gb):
                        wait_idx(ib2)
                        gather(ib2, gb)

                    pl.when(j + ngb < rows_per_tile)(nxt_gather)
                return 0

            lax.fori_loop(0, rows_per_tile // nib, outer, 0)
            plsc.subcore_barrier()
            for q in range(wb_chunks):
                sl = pl.ds(s * n_per_tile + q * LANE, LANE)
                pltpu.sync_copy(acc.at[sl], rows_v[0])
                pltpu.sync_copy(rows_v[0], out.at[sl])

        @pl.when(c == 0)
        def _():
            run(tab_lo, out_lo)

        @pl.when(c == 1)
        def _():
            run(tab_hi, out_hi)

    return agg_kernel


def _tc_scale(deg0, deg1, x_pad, half):
    n_pad, fin = x_pad.shape
    blk = 1024
    grid = (n_pad // blk,)

    def body(d0, d1, x, lo, hi):
        dinv = lax.rsqrt(d0[...] + d1[...] + 1.0)
        xs = (x[...] * dinv[:, None]).astype(AGG_DT)
        lo[...] = xs[:, :half]
        hi[...] = xs[:, half:]

    return pl.pallas_call(
        body,
        grid=grid,
        in_specs=[
            pl.BlockSpec((blk,), lambda i: (i,)),
            pl.BlockSpec((blk,), lambda i: (i,)),
            pl.BlockSpec((blk, fin), lambda i: (i, 0)),
        ],
        out_specs=[
            pl.BlockSpec((blk, half), lambda i: (i, 0)),
            pl.BlockSpec((blk, half), lambda i: (i, 0)),
        ],
        out_shape=[
            jax.ShapeDtypeStruct((n_pad, half), AGG_DT),
            jax.ShapeDtypeStruct((n_pad, half), AGG_DT),
        ],
    )(deg0, deg1, x_pad)


def _tc_layer1(deg0, deg1, x_pad, u_lo, u_hi, W1, b1, W2, half):
    n_pad, fin = x_pad.shape
    fmid = W1.shape[1]
    blk = 1024
    grid = (n_pad // blk,)

    def body(d0, d1, x, ulo, uhi, w1, bb1, w2, t_out, tslo, tshi):
        dinv = lax.rsqrt(d0[...] + d1[...] + 1.0)
        u = jnp.concatenate([ulo[...], uhi[...]], axis=1).astype(F32)
        z = u * dinv[:, None] + x[...] * (dinv * dinv)[:, None]
        bf = jnp.bfloat16
        h = jnp.maximum(
            jnp.dot(z.astype(bf), w1[...].astype(bf),
                    preferred_element_type=F32) + bb1[...][None, :],
            0.0,
        )
        t = jnp.dot(h.astype(bf), w2[...].astype(bf),
                    preferred_element_type=F32)
        t_out[...] = t
        ts = (t * dinv[:, None]).astype(AGG_DT)
        tslo[...] = ts[:, :half]
        tshi[...] = ts[:, half:]

    return pl.pallas_call(
        body,
        grid=grid,
        in_specs=[
            pl.BlockSpec((blk,), lambda i: (i,)),
            pl.BlockSpec((blk,), lambda i: (i,)),
            pl.BlockSpec((blk, fin), lambda i: (i, 0)),
            pl.BlockSpec((blk, half), lambda i: (i, 0)),
            pl.BlockSpec((blk, half), lambda i: (i, 0)),
            pl.BlockSpec((fin, fmid), lambda i: (0, 0)),
            pl.BlockSpec((fmid,), lambda i: (0,)),
            pl.BlockSpec((fmid, fin), lambda i: (0, 0)),
        ],
        out_specs=[
            pl.BlockSpec((blk, fin), lambda i: (i, 0)),
            pl.BlockSpec((blk, half), lambda i: (i, 0)),
            pl.BlockSpec((blk, half), lambda i: (i, 0)),
        ],
        out_shape=[
            jax.ShapeDtypeStruct((n_pad, fin), F32),
            jax.ShapeDtypeStruct((n_pad, half), AGG_DT),
            jax.ShapeDtypeStruct((n_pad, half), AGG_DT),
        ],
    )(deg0, deg1, x_pad, u_lo, u_hi, W1, b1, W2)


def _tc_finish(deg0, deg1, t, u_lo, u_hi, b2):
    n_pad, fout = t.shape
    half = fout // 2
    blk = 1024
    grid = (n_pad // blk,)

    def body(d0, d1, tt, ulo, uhi, bb2, out):
        dinv = lax.rsqrt(d0[...] + d1[...] + 1.0)
        u = jnp.concatenate([ulo[...], uhi[...]], axis=1).astype(F32)
        z = u * dinv[:, None] + tt[...] * (dinv * dinv)[:, None] + bb2[...][None, :]
        r = jnp.maximum(z, 0.0)
        m = jnp.max(r, axis=1, keepdims=True)
        lse = m + jnp.log(jnp.sum(jnp.exp(r - m), axis=1, keepdims=True))
        out[...] = r - lse

    return pl.pallas_call(
        body,
        grid=grid,
        in_specs=[
            pl.BlockSpec((blk,), lambda i: (i,)),
            pl.BlockSpec((blk,), lambda i: (i,)),
            pl.BlockSpec((blk, fout), lambda i: (i, 0)),
            pl.BlockSpec((blk, half), lambda i: (i, 0)),
            pl.BlockSpec((blk, half), lambda i: (i, 0)),
            pl.BlockSpec((fout,), lambda i: (0,)),
        ],
        out_specs=pl.BlockSpec((blk, fout), lambda i: (i, 0)),
        out_shape=jax.ShapeDtypeStruct((n_pad, fout), F32),
    )(deg0, deg1, t, u_lo, u_hi, b2)


def kernel(x, edge_index, W1, b1, W2, b2):
    n, fin = x.shape
    half = fin // 2
    e = edge_index.shape[1]

    n_pad = ((n + 1 + 1023) // 1024) * 1024
    e_pad = ((e + NC * NS * LANE - 1) // (NC * NS * LANE)) * (NC * NS * LANE)

    ei = edge_index.astype(jnp.int32)
    pad = jnp.full((e_pad - e,), n, jnp.int32)
    src2d = jnp.concatenate([ei[0], pad]).reshape(e_pad // LANE, LANE)
    dst2d = jnp.concatenate([ei[1], pad]).reshape(e_pad // LANE, LANE)
    edg3d = jnp.stack([src2d, dst2d], axis=1)
    x_pad = jnp.pad(x, ((0, n_pad - n), (0, 0)))
    zeros1 = jnp.zeros((n_pad,), F32)
    zeros2 = jnp.zeros((n_pad, half), AGG_DT)

    deg0, deg1 = _make_deg(n_pad, e_pad // LANE)(dst2d, zeros1)
    xs_lo, xs_hi = _tc_scale(deg0, deg1, x_pad, half)
    agg = _make_agg(n_pad, e_pad // LANE, half)
    u1_lo, u1_hi = agg(edg3d, xs_lo, xs_hi, zeros2)
    t, ts_lo, ts_hi = _tc_layer1(deg0, deg1, x_pad, u1_lo, u1_hi, W1, b1, W2, half)
    u2_lo, u2_hi = agg(edg3d, ts_lo, ts_hi, zeros2)
    o = _tc_finish(deg0, deg1, t, u2_lo, u2_hi, b2)
    return o[:n]
